# Initial kernel scaffold; baseline (speedup 1.0000x reference)
import jax
import jax.numpy as jnp
from jax.experimental import pallas as pl


def kernel(sweep, map, W_sweep, b_sweep, gamma_sweep, beta_sweep, mean_sweep, var_sweep, W_map, b_map, gamma_map, beta_map, mean_map, var_map, W_backbone, b_backbone):
    raise NotImplementedError("write your pallas kernel here")


# speedup vs baseline: 3.9117x; 3.9117x over previous
import numpy as np
import jax
import jax.numpy as jnp
from jax import lax
from jax.experimental import pallas as pl
from jax.experimental.pallas import tpu as pltpu

H = 282
W = 282
HW = H * W
P = 12000
N = 32
PT = 12
ZT = P // PT
MT = ZT * N
EDGES = np.arange(-22, 22, 0.16).astype(np.float32)
NE = len(EDGES)
CR = 16
CC = P // CR


def _pfn_body(x_ref, w_ref, s_ref, t_ref, wb_ref, o_ref):
    x = x_ref[0]
    y = lax.dot_general(x, w_ref[...], (((0,), (1,)), ((), ())))
    y = y * s_ref[...] + t_ref[...]
    y = jnp.maximum(y, 0.0)
    z = jnp.max(y.reshape(ZT, N, 64), axis=1)
    o_ref[0] = lax.dot_general(z, wb_ref[...], (((1,), (1,)), ((), ())))


def _pfn_call(xf, w, s, t, wb):
    b = xf.shape[0]
    return pl.pallas_call(
        _pfn_body,
        grid=(b, PT),
        in_specs=[
            pl.BlockSpec((1, 8, MT), lambda i, j: (i, 0, j)),
            pl.BlockSpec((64, 8), lambda i, j: (0, 0)),
            pl.BlockSpec((1, 64), lambda i, j: (0, 0)),
            pl.BlockSpec((1, 64), lambda i, j: (0, 0)),
            pl.BlockSpec((64, 64), lambda i, j: (0, 0)),
        ],
        out_specs=pl.BlockSpec((1, ZT, 64), lambda i, j: (i, j, 0)),
        out_shape=jax.ShapeDtypeStruct((b, P, 64), jnp.float32),
    )(xf, w, s, t, wb)


def _cells_body(c_ref, o_ref):
    outs = []
    for a in range(2):
        x = c_ref[0, 2 * a]
        y = c_ref[0, 2 * a + 1]
        ax = jnp.zeros((CR, CC), dtype=jnp.int32)
        ay = jnp.zeros((CR, CC), dtype=jnp.int32)
        for k in range(NE):
            e = float(EDGES[k])
            ax = ax + (x >= e).astype(jnp.int32)
            ay = ay + (y >= e).astype(jnp.int32)
        xg = jnp.clip(ax - 1, 0, W - 1)
        yg = jnp.clip(ay - 1, 0, H - 1)
        outs.append(yg * W + xg)
    o_ref[0] = jnp.stack(outs, axis=0)


def _cells_call(coords):
    b = coords.shape[0]
    return pl.pallas_call(
        _cells_body,
        grid=(b,),
        in_specs=[pl.BlockSpec((1, 4, CR, CC), lambda i: (i, 0, 0, 0))],
        out_specs=pl.BlockSpec((1, 2, CR, CC), lambda i: (i, 0, 0, 0)),
        out_shape=jax.ShapeDtypeStruct((b, 2, CR, CC), jnp.int32),
    )(coords)


def _scatter_body(cells_ref, ts_hbm, tm_hbm, bias_ref, o_ref,
                  canvas_ref, ts_ref, tm_ref, wmap_ref, sem):
    pid = pl.program_id(0)
    cp_ts = pltpu.make_async_copy(ts_hbm.at[pid], ts_ref.at[0], sem)
    cp_ts.start()
    cp_tm = pltpu.make_async_copy(tm_hbm.at[pid], tm_ref.at[0], sem)
    cp_tm.start()
    bias = bias_ref[...]
    canvas_ref[...] = jnp.broadcast_to(bias, (HW, 64))
    cp_ts.wait()
    cp_tm.wait()

    def loop1(p, carry):
        c = cells_ref[0, 0, p]
        canvas_ref[pl.ds(c, 1), :] = ts_ref[0, pl.ds(p, 1), :] + bias
        return carry

    lax.fori_loop(0, P, loop1, 0)

    def loop2(p, carry):
        wmap_ref[cells_ref[0, 1, p]] = p
        return carry

    lax.fori_loop(0, P, loop2, 0)

    def loop3(p, carry):
        c = cells_ref[0, 1, p]

        @pl.when(wmap_ref[c] == p)
        def _add():
            canvas_ref[pl.ds(c, 1), :] = (
                canvas_ref[pl.ds(c, 1), :] + tm_ref[0, pl.ds(p, 1), :]
            )

        return carry

    lax.fori_loop(0, P, loop3, 0)

    cp = pltpu.make_async_copy(canvas_ref, o_ref.at[pid], sem)
    cp.start()
    cp.wait()


def _scatter_call(cells, ts, tm, bias2d):
    b = cells.shape[0]
    return pl.pallas_call(
        _scatter_body,
        grid=(b,),
        in_specs=[
            pl.BlockSpec((1, 2, P), lambda i: (i, 0, 0), memory_space=pltpu.SMEM),
            pl.BlockSpec(memory_space=pl.ANY),
            pl.BlockSpec(memory_space=pl.ANY),
            pl.BlockSpec((1, 64), lambda i: (0, 0)),
        ],
        out_specs=pl.BlockSpec(memory_space=pl.ANY),
        out_shape=jax.ShapeDtypeStruct((b, HW, 64), jnp.float32),
        scratch_shapes=[
            pltpu.VMEM((HW, 64), jnp.float32),
            pltpu.VMEM((1, P, 64), jnp.float32),
            pltpu.VMEM((1, P, 64), jnp.float32),
            pltpu.SMEM((HW,), jnp.int32),
            pltpu.SemaphoreType.DMA,
        ],
    )(cells, ts, tm, bias2d)


def kernel(sweep, map, W_sweep, b_sweep, gamma_sweep, beta_sweep,
           mean_sweep, var_sweep, W_map, b_map, gamma_map, beta_map,
           mean_map, var_map, W_backbone, b_backbone):
    b = sweep.shape[0]
    s_s = gamma_sweep / jnp.sqrt(var_sweep + 1e-5)
    t_s = (b_sweep - mean_sweep) * s_s + beta_sweep
    s_m = gamma_map / jnp.sqrt(var_map + 1e-5)
    t_m = (b_map - mean_map) * s_m + beta_map
    wb1 = W_backbone[:, :64]
    wb2 = W_backbone[:, 64:]

    ts = _pfn_call(sweep.reshape(b, 8, P * N), W_sweep,
                   s_s.reshape(1, 64), t_s.reshape(1, 64), wb1)
    tm = _pfn_call(map.reshape(b, 8, P * N), W_map,
                   s_m.reshape(1, 64), t_m.reshape(1, 64), wb2)

    coords = jnp.concatenate([sweep[:, 0:2, :, 0], map[:, 0:2, :, 0]],
                             axis=1).reshape(b, 4, CR, CC)
    cells = _cells_call(coords).reshape(b, 2, P)

    out_t = _scatter_call(cells, ts, tm, b_backbone.reshape(1, 64))
    return out_t.transpose(0, 2, 1).reshape(b, 64, H, W)
